# baseline (speedup 1.0000x reference)
import functools

import jax
import jax.numpy as jnp
from jax.experimental import pallas as pl

THRESHOLD = 0.5
TEMPERATURE = 0.1

_HI = jax.lax.Precision.HIGHEST


def _stage1_kernel(nch, nc1, lab_ref, oo_ref, fT_ref, fS_ref,
                   sT_ref, cT_ref, sS_ref, cS_ref):
    p = pl.program_id(1)

    @pl.when(p == 0)
    def _init():
        sT_ref[...] = jnp.zeros_like(sT_ref)
        cT_ref[...] = jnp.zeros_like(cT_ref)
        sS_ref[...] = jnp.zeros_like(sS_ref)
        cS_ref[...] = jnp.zeros_like(cS_ref)

    lab = lab_ref[0].astype(jnp.float32)
    outs = oo_ref[0]

    best = jnp.where(outs[0] < THRESHOLD, 0.0, outs[0])
    besti = jnp.zeros((16, 512), jnp.float32)
    for c in range(1, nch):
        v = jnp.where(outs[c] < THRESHOLD, 0.0, outs[c])
        upd = v > best
        besti = jnp.where(upd, jnp.float32(c), besti)
        best = jnp.where(upd, v, best)

    selC = (jax.lax.broadcasted_iota(jnp.int32, (512, 128), 0)
            == 4 * jax.lax.broadcasted_iota(jnp.int32, (512, 128), 1)
            ).astype(jnp.float32)

    lab_d = jax.lax.dot(lab, selC)
    arg_d = jax.lax.dot(besti, selC)
    bgr = lab_d == 0.0
    segT = jnp.where(bgr, arg_d, 0.0)
    segS = segT + lab_d

    iota_c = jax.lax.broadcasted_iota(jnp.int32, (128, nc1), 1
                                      ).astype(jnp.float32)
    ones_row = jnp.ones((1, 2048), jnp.float32)

    def accum(seg, f_ref, s_ref, c_ref):
        segT_ = jnp.transpose(seg)
        onehot = jnp.concatenate(
            [(segT_[:, r:r + 1] == iota_c).astype(jnp.float32)
             for r in range(16)], axis=0)
        feat = f_ref[0]
        oh_bf = onehot.astype(jnp.bfloat16)
        f_hi = feat.astype(jnp.bfloat16)
        f_lo = (feat - f_hi.astype(jnp.float32)).astype(jnp.bfloat16)
        s_ref[0] += (jax.lax.dot(f_hi, oh_bf,
                                 preferred_element_type=jnp.float32)
                     + jax.lax.dot(f_lo, oh_bf,
                                   preferred_element_type=jnp.float32))
        c_ref[0] += jax.lax.dot(ones_row, onehot)

    accum(segT, fT_ref, sT_ref, cT_ref)
    accum(segS, fS_ref, sS_ref, cS_ref)


def _stage2_kernel(nc1, noc, sT_ref, cT_ref, sS_ref, cS_ref,
                   proto_ref, sc_ref, loss_ref):
    B = sT_ref.shape[0]
    NS = B + 1
    NR = NS * nc1
    nct = sc_ref[0, 0]
    noct = sc_ref[0, 1]

    cT = cT_ref[:, 0, :]
    cS = cS_ref[:, 0, :]

    def norm_rows(m):
        n2 = jnp.sum(m * m, axis=1, keepdims=True)
        return m / jnp.maximum(jnp.sqrt(n2), 1e-12)

    def mean_rows(s_ref, cnt):
        rows = []
        for b in range(B):
            m = s_ref[b] / jnp.maximum(cnt[b], 1.0)[None, :]
            rows.append(norm_rows(m.T))
        return rows

    protoN = norm_rows(proto_ref[...])
    AS = jnp.concatenate(mean_rows(sS_ref, cS) + [protoN], axis=0)
    AT = jnp.concatenate(mean_rows(sT_ref, cT) + [protoN], axis=0)

    nt = (((1,), (1,)), ((), ()))
    ES = jnp.exp(jax.lax.dot_general(AS, AS, nt, precision=_HI)
                 / TEMPERATURE)
    ET = jnp.exp(jax.lax.dot_general(AT, AS, nt, precision=_HI)
                 / TEMPERATURE)

    presS = (cS > 0).astype(jnp.float32)
    presT = (cT > 0).astype(jnp.float32)
    ones21 = jnp.ones((1, nc1), jnp.float32)
    validS = jnp.concatenate([presS, ones21], axis=0)
    validT = jnp.concatenate([presT, ones21], axis=0)
    cS_tot = 1.0 + jnp.sum(presS, axis=0, keepdims=True)
    cT_tot = 1.0 + jnp.sum(presT, axis=0, keepdims=True)

    Eb = (jax.lax.broadcasted_iota(jnp.int32, (NR, NS), 0) // nc1
          == jax.lax.broadcasted_iota(jnp.int32, (NR, NS), 1)
          ).astype(jnp.float32)
    Ec = (jax.lax.broadcasted_iota(jnp.int32, (NR, nc1), 0) % nc1
          == jax.lax.broadcasted_iota(jnp.int32, (NR, nc1), 1)
          ).astype(jnp.float32)
    Fb = (jax.lax.broadcasted_iota(jnp.int32, (NS, NR), 1) // nc1
          == jax.lax.broadcasted_iota(jnp.int32, (NS, NR), 0)
          ).astype(jnp.float32)
    Fc = (jax.lax.broadcasted_iota(jnp.int32, (nc1, NR), 1) % nc1
          == jax.lax.broadcasted_iota(jnp.int32, (nc1, NR), 0)
          ).astype(jnp.float32)

    def row_lut(tab):
        return jnp.sum(jax.lax.dot(Eb, tab, precision=_HI) * Ec,
                       axis=1, keepdims=True)

    def col_lut(tab):
        return jnp.sum(jax.lax.dot(tab, Fc, precision=_HI) * Fb,
                       axis=0, keepdims=True)

    ri = jax.lax.broadcasted_iota(jnp.int32, (NR, NR), 0)
    ci = jax.lax.broadcasted_iota(jnp.int32, (NR, NR), 1)
    rc = ri % nc1
    cc = ci % nc1

    w_row = row_lut(validS / cS_tot)
    mrow = ((rc >= 1) & (rc != cc)
            & (rc.astype(jnp.float32) <= nct)).astype(jnp.float32)
    snt = jnp.sum(ES * w_row * mrow, axis=0, keepdims=True)

    vT_row = row_lut(validT)
    bc = jnp.broadcast_to
    cTtot_col = col_lut(bc(cT_tot, (NS, nc1)))
    dj_col = col_lut(bc(jnp.maximum(cS_tot - 1.0, 1.0), (NS, nc1)))
    fval_col = col_lut(jnp.concatenate(
        [presS, jnp.zeros((1, nc1), jnp.float32)], axis=0))
    ci1 = jax.lax.broadcasted_iota(jnp.int32, (1, NR), 1)
    ccol = ci1 % nc1
    fmask = fval_col * ((ccol >= 1) & (ccol <= noc)
                        & (ccol.astype(jnp.float32) <= noct)
                        ).astype(jnp.float32)

    terms = jnp.log(ET / (snt + ET))
    pmask = vT_row * (rc == cc).astype(jnp.float32)
    loss = -jnp.sum(terms * pmask * (fmask / cTtot_col / dj_col),
                    keepdims=True)

    ci21 = jax.lax.broadcasted_iota(jnp.int32, (1, nc1), 1)
    emask = ((ci21 >= 1) & (ci21 <= noc)
             & (ci21.astype(jnp.float32) <= noct)
             & (cS_tot > 1.0)).astype(jnp.float32)
    cnt_exist = jnp.sum(emask, keepdims=True)
    loss_ref[...] = jnp.where(cnt_exist > 0.0, loss / cnt_exist, loss)


def kernel(labels, features_old, features, outputs_old, prototypes,
           num_class, num_old_class):
    B, C, H, W = features.shape
    nch = outputs_old.shape[1]
    noc = nch - 1
    nc1 = prototypes.shape[0]
    NPIX = H * W
    CHUNK = 2048
    K = NPIX // CHUNK

    lab_v = labels.astype(jnp.int32).reshape(B, 128, 2048)
    oo_v = outputs_old.reshape(B, nch, 128, 2048)
    fT = features_old.reshape(B, C, NPIX)
    fS = features.reshape(B, C, NPIX)
    scal = jnp.stack([jnp.asarray(num_class, jnp.float32),
                      jnp.asarray(num_old_class, jnp.float32)]).reshape(1, 2)

    sT, cT, sS, cS = pl.pallas_call(
        functools.partial(_stage1_kernel, nch, nc1),
        grid=(B, K),
        in_specs=[
            pl.BlockSpec((1, 16, 512), lambda b, p: (b, p, 0)),
            pl.BlockSpec((1, nch, 16, 512), lambda b, p: (b, 0, p, 0)),
            pl.BlockSpec((1, C, CHUNK), lambda b, p: (b, 0, p)),
            pl.BlockSpec((1, C, CHUNK), lambda b, p: (b, 0, p)),
        ],
        out_specs=[
            pl.BlockSpec((1, C, nc1), lambda b, p: (b, 0, 0)),
            pl.BlockSpec((1, 1, nc1), lambda b, p: (b, 0, 0)),
            pl.BlockSpec((1, C, nc1), lambda b, p: (b, 0, 0)),
            pl.BlockSpec((1, 1, nc1), lambda b, p: (b, 0, 0)),
        ],
        out_shape=[
            jax.ShapeDtypeStruct((B, C, nc1), jnp.float32),
            jax.ShapeDtypeStruct((B, 1, nc1), jnp.float32),
            jax.ShapeDtypeStruct((B, C, nc1), jnp.float32),
            jax.ShapeDtypeStruct((B, 1, nc1), jnp.float32),
        ],
    )(lab_v, oo_v, fT, fS)

    loss = pl.pallas_call(
        functools.partial(_stage2_kernel, nc1, noc),
        in_specs=[
            pl.BlockSpec((B, C, nc1), lambda: (0, 0, 0)),
            pl.BlockSpec((B, 1, nc1), lambda: (0, 0, 0)),
            pl.BlockSpec((B, C, nc1), lambda: (0, 0, 0)),
            pl.BlockSpec((B, 1, nc1), lambda: (0, 0, 0)),
            pl.BlockSpec((nc1, C), lambda: (0, 0)),
            pl.BlockSpec((1, 2), lambda: (0, 0)),
        ],
        out_specs=pl.BlockSpec((1, 1), lambda: (0, 0)),
        out_shape=jax.ShapeDtypeStruct((1, 1), jnp.float32),
    )(sT, cT, sS, cS, prototypes, scal)

    return loss[0, 0]


# speedup vs baseline: 1.3413x; 1.3413x over previous
import functools

import jax
import jax.numpy as jnp
from jax.experimental import pallas as pl

THRESHOLD = 0.5
TEMPERATURE = 0.1

_HI = jax.lax.Precision.HIGHEST


def _stage1_kernel(nch, nc1, R, lab_ref, oo_ref, fT_ref, fS_ref,
                   sT_ref, cT_ref, sS_ref, cS_ref):
    RF = 4 * R
    p = pl.program_id(1)

    @pl.when(p == 0)
    def _init():
        sT_ref[...] = jnp.zeros_like(sT_ref)
        cT_ref[...] = jnp.zeros_like(cT_ref)
        sS_ref[...] = jnp.zeros_like(sS_ref)
        cS_ref[...] = jnp.zeros_like(cS_ref)

    lab = lab_ref[0].astype(jnp.float32)
    outs = oo_ref[0]

    best = jnp.where(outs[0] < THRESHOLD, 0.0, outs[0])
    besti = jnp.zeros((RF, 512), jnp.float32)
    for c in range(1, nch):
        v = jnp.where(outs[c] < THRESHOLD, 0.0, outs[c])
        upd = v > best
        besti = jnp.where(upd, jnp.float32(c), besti)
        best = jnp.where(upd, v, best)

    selR = (jax.lax.broadcasted_iota(jnp.int32, (R, RF), 1)
            == 4 * jax.lax.broadcasted_iota(jnp.int32, (R, RF), 0)
            ).astype(jnp.float32)
    selC = (jax.lax.broadcasted_iota(jnp.int32, (512, 128), 0)
            == 4 * jax.lax.broadcasted_iota(jnp.int32, (512, 128), 1)
            ).astype(jnp.float32)

    def down(x):
        return jax.lax.dot(jax.lax.dot(selR, x), selC)

    lab_d = down(lab)
    arg_d = down(besti)
    bgr = lab_d == 0.0
    segT = jnp.where(bgr, arg_d, 0.0)
    segS = segT + lab_d

    iota_c = jax.lax.broadcasted_iota(jnp.int32, (128, nc1), 1
                                      ).astype(jnp.float32)
    ones_row = jnp.ones((1, 128 * R), jnp.float32)

    def accum(seg, f_ref, s_ref, c_ref):
        segT_ = jnp.transpose(seg)
        onehot = jnp.concatenate(
            [(segT_[:, r:r + 1] == iota_c).astype(jnp.float32)
             for r in range(R)], axis=0)
        feat = f_ref[0]
        oh_bf = onehot.astype(jnp.bfloat16)
        f_hi = feat.astype(jnp.bfloat16)
        f_lo = (feat - f_hi.astype(jnp.float32)).astype(jnp.bfloat16)
        s_ref[0] += (jax.lax.dot(f_hi, oh_bf,
                                 preferred_element_type=jnp.float32)
                     + jax.lax.dot(f_lo, oh_bf,
                                   preferred_element_type=jnp.float32))
        c_ref[0] += jax.lax.dot(ones_row, onehot)

    accum(segT, fT_ref, sT_ref, cT_ref)
    accum(segS, fS_ref, sS_ref, cS_ref)


def _stage2_kernel(nc1, noc, sT_ref, cT_ref, sS_ref, cS_ref,
                   proto_ref, sc_ref, loss_ref):
    B = sT_ref.shape[0]
    NS = B + 1
    NR = NS * nc1
    nct = sc_ref[0, 0]
    noct = sc_ref[0, 1]

    cT = cT_ref[:, 0, :]
    cS = cS_ref[:, 0, :]

    def norm_rows(m):
        n2 = jnp.sum(m * m, axis=1, keepdims=True)
        return m / jnp.maximum(jnp.sqrt(n2), 1e-12)

    def mean_rows(s_ref, cnt):
        rows = []
        for b in range(B):
            m = s_ref[b] / jnp.maximum(cnt[b], 1.0)[None, :]
            rows.append(norm_rows(m.T))
        return rows

    protoN = norm_rows(proto_ref[...])
    AS = jnp.concatenate(mean_rows(sS_ref, cS) + [protoN], axis=0)
    AT = jnp.concatenate(mean_rows(sT_ref, cT) + [protoN], axis=0)

    nt = (((1,), (1,)), ((), ()))
    ES = jnp.exp(jax.lax.dot_general(AS, AS, nt, precision=_HI)
                 / TEMPERATURE)
    ET = jnp.exp(jax.lax.dot_general(AT, AS, nt, precision=_HI)
                 / TEMPERATURE)

    presS = (cS > 0).astype(jnp.float32)
    presT = (cT > 0).astype(jnp.float32)
    ones21 = jnp.ones((1, nc1), jnp.float32)
    validS = jnp.concatenate([presS, ones21], axis=0)
    validT = jnp.concatenate([presT, ones21], axis=0)
    cS_tot = 1.0 + jnp.sum(presS, axis=0, keepdims=True)
    cT_tot = 1.0 + jnp.sum(presT, axis=0, keepdims=True)

    Eb = (jax.lax.broadcasted_iota(jnp.int32, (NR, NS), 0) // nc1
          == jax.lax.broadcasted_iota(jnp.int32, (NR, NS), 1)
          ).astype(jnp.float32)
    Ec = (jax.lax.broadcasted_iota(jnp.int32, (NR, nc1), 0) % nc1
          == jax.lax.broadcasted_iota(jnp.int32, (NR, nc1), 1)
          ).astype(jnp.float32)
    Fb = (jax.lax.broadcasted_iota(jnp.int32, (NS, NR), 1) // nc1
          == jax.lax.broadcasted_iota(jnp.int32, (NS, NR), 0)
          ).astype(jnp.float32)
    Fc = (jax.lax.broadcasted_iota(jnp.int32, (nc1, NR), 1) % nc1
          == jax.lax.broadcasted_iota(jnp.int32, (nc1, NR), 0)
          ).astype(jnp.float32)

    def row_lut(tab):
        return jnp.sum(jax.lax.dot(Eb, tab, precision=_HI) * Ec,
                       axis=1, keepdims=True)

    def col_lut(tab):
        return jnp.sum(jax.lax.dot(tab, Fc, precision=_HI) * Fb,
                       axis=0, keepdims=True)

    ri = jax.lax.broadcasted_iota(jnp.int32, (NR, NR), 0)
    ci = jax.lax.broadcasted_iota(jnp.int32, (NR, NR), 1)
    rc = ri % nc1
    cc = ci % nc1

    w_row = row_lut(validS / cS_tot)
    mrow = ((rc >= 1) & (rc != cc)
            & (rc.astype(jnp.float32) <= nct)).astype(jnp.float32)
    snt = jnp.sum(ES * w_row * mrow, axis=0, keepdims=True)

    vT_row = row_lut(validT)
    bc = jnp.broadcast_to
    cTtot_col = col_lut(bc(cT_tot, (NS, nc1)))
    dj_col = col_lut(bc(jnp.maximum(cS_tot - 1.0, 1.0), (NS, nc1)))
    fval_col = col_lut(jnp.concatenate(
        [presS, jnp.zeros((1, nc1), jnp.float32)], axis=0))
    ci1 = jax.lax.broadcasted_iota(jnp.int32, (1, NR), 1)
    ccol = ci1 % nc1
    fmask = fval_col * ((ccol >= 1) & (ccol <= noc)
                        & (ccol.astype(jnp.float32) <= noct)
                        ).astype(jnp.float32)

    terms = jnp.log(ET / (snt + ET))
    pmask = vT_row * (rc == cc).astype(jnp.float32)
    loss = -jnp.sum(terms * pmask * (fmask / cTtot_col / dj_col),
                    keepdims=True)

    ci21 = jax.lax.broadcasted_iota(jnp.int32, (1, nc1), 1)
    emask = ((ci21 >= 1) & (ci21 <= noc)
             & (ci21.astype(jnp.float32) <= noct)
             & (cS_tot > 1.0)).astype(jnp.float32)
    cnt_exist = jnp.sum(emask, keepdims=True)
    loss_ref[...] = jnp.where(cnt_exist > 0.0, loss / cnt_exist, loss)


def kernel(labels, features_old, features, outputs_old, prototypes,
           num_class, num_old_class):
    B, C, H, W = features.shape
    nch = outputs_old.shape[1]
    noc = nch - 1
    nc1 = prototypes.shape[0]
    NPIX = H * W
    R = 32
    CHUNK = 128 * R
    K = NPIX // CHUNK

    lab_v = labels.astype(jnp.int32)
    fT = features_old.reshape(B, C, NPIX)
    fS = features.reshape(B, C, NPIX)
    scal = jnp.stack([jnp.asarray(num_class, jnp.float32),
                      jnp.asarray(num_old_class, jnp.float32)]).reshape(1, 2)

    sT, cT, sS, cS = pl.pallas_call(
        functools.partial(_stage1_kernel, nch, nc1, R),
        grid=(B, K),
        in_specs=[
            pl.BlockSpec((1, 4 * R, 512), lambda b, p: (b, p, 0)),
            pl.BlockSpec((1, nch, 4 * R, 512), lambda b, p: (b, 0, p, 0)),
            pl.BlockSpec((1, C, CHUNK), lambda b, p: (b, 0, p)),
            pl.BlockSpec((1, C, CHUNK), lambda b, p: (b, 0, p)),
        ],
        out_specs=[
            pl.BlockSpec((1, C, nc1), lambda b, p: (b, 0, 0)),
            pl.BlockSpec((1, 1, nc1), lambda b, p: (b, 0, 0)),
            pl.BlockSpec((1, C, nc1), lambda b, p: (b, 0, 0)),
            pl.BlockSpec((1, 1, nc1), lambda b, p: (b, 0, 0)),
        ],
        out_shape=[
            jax.ShapeDtypeStruct((B, C, nc1), jnp.float32),
            jax.ShapeDtypeStruct((B, 1, nc1), jnp.float32),
            jax.ShapeDtypeStruct((B, C, nc1), jnp.float32),
            jax.ShapeDtypeStruct((B, 1, nc1), jnp.float32),
        ],
    )(lab_v, outputs_old, fT, fS)

    loss = pl.pallas_call(
        functools.partial(_stage2_kernel, nc1, noc),
        in_specs=[
            pl.BlockSpec((B, C, nc1), lambda: (0, 0, 0)),
            pl.BlockSpec((B, 1, nc1), lambda: (0, 0, 0)),
            pl.BlockSpec((B, C, nc1), lambda: (0, 0, 0)),
            pl.BlockSpec((B, 1, nc1), lambda: (0, 0, 0)),
            pl.BlockSpec((nc1, C), lambda: (0, 0)),
            pl.BlockSpec((1, 2), lambda: (0, 0)),
        ],
        out_specs=pl.BlockSpec((1, 1), lambda: (0, 0)),
        out_shape=jax.ShapeDtypeStruct((1, 1), jnp.float32),
    )(sT, cT, sS, cS, prototypes, scal)

    return loss[0, 0]


# speedup vs baseline: 1.3618x; 1.0153x over previous
import functools

import jax
import jax.numpy as jnp
from jax.experimental import pallas as pl

THRESHOLD = 0.5
TEMPERATURE = 0.1

_HI = jax.lax.Precision.HIGHEST


def _stage1_kernel(nch, nc1, R, lab_ref, oo_ref, fT_ref, fS_ref,
                   sT_ref, cT_ref, sS_ref, cS_ref):
    RF = 4 * R
    p = pl.program_id(1)

    @pl.when(p == 0)
    def _init():
        sT_ref[...] = jnp.zeros_like(sT_ref)
        cT_ref[...] = jnp.zeros_like(cT_ref)
        sS_ref[...] = jnp.zeros_like(sS_ref)
        cS_ref[...] = jnp.zeros_like(cS_ref)

    lab = lab_ref[0].astype(jnp.float32)
    outs = oo_ref[0]

    best = jnp.where(outs[0] < THRESHOLD, 0.0, outs[0])
    besti = jnp.zeros((RF, 512), jnp.float32)
    for c in range(1, nch):
        v = jnp.where(outs[c] < THRESHOLD, 0.0, outs[c])
        upd = v > best
        besti = jnp.where(upd, jnp.float32(c), besti)
        best = jnp.where(upd, v, best)

    selR = (jax.lax.broadcasted_iota(jnp.int32, (R, RF), 1)
            == 4 * jax.lax.broadcasted_iota(jnp.int32, (R, RF), 0)
            ).astype(jnp.float32)
    selC = (jax.lax.broadcasted_iota(jnp.int32, (512, 128), 0)
            == 4 * jax.lax.broadcasted_iota(jnp.int32, (512, 128), 1)
            ).astype(jnp.float32)

    def down(x):
        return jax.lax.dot(jax.lax.dot(selR, x), selC)

    lab_d = down(lab)
    arg_d = down(besti)
    bgr = lab_d == 0.0
    segT = jnp.where(bgr, arg_d, 0.0)
    segS = segT + lab_d

    iota_c = jax.lax.broadcasted_iota(jnp.int32, (128, nc1), 1
                                      ).astype(jnp.float32)
    ones_row = jnp.ones((1, 128 * R), jnp.float32)

    def accum(seg, f_ref, s_ref, c_ref):
        segT_ = jnp.transpose(seg)
        onehot = jnp.concatenate(
            [(segT_[:, r:r + 1] == iota_c).astype(jnp.float32)
             for r in range(R)], axis=0)
        feat = f_ref[0]
        oh_bf = onehot.astype(jnp.bfloat16)
        f_hi = feat.astype(jnp.bfloat16)
        f_lo = (feat - f_hi.astype(jnp.float32)).astype(jnp.bfloat16)
        s_ref[0] += (jax.lax.dot(f_hi, oh_bf,
                                 preferred_element_type=jnp.float32)
                     + jax.lax.dot(f_lo, oh_bf,
                                   preferred_element_type=jnp.float32))
        c_ref[0] += jax.lax.dot(ones_row, onehot)

    accum(segT, fT_ref, sT_ref, cT_ref)
    accum(segS, fS_ref, sS_ref, cS_ref)


def _stage2_kernel(nc1, noc, sT_ref, cT_ref, sS_ref, cS_ref,
                   proto_ref, sc_ref, loss_ref):
    B = sT_ref.shape[0]
    NS = B + 1
    NR = NS * nc1
    nct = sc_ref[0, 0]
    noct = sc_ref[0, 1]

    cT = cT_ref[:, 0, :]
    cS = cS_ref[:, 0, :]

    def norm_rows(m):
        n2 = jnp.sum(m * m, axis=1, keepdims=True)
        return m / jnp.maximum(jnp.sqrt(n2), 1e-12)

    def mean_rows(s_ref, cnt):
        rows = []
        for b in range(B):
            m = s_ref[b] / jnp.maximum(cnt[b], 1.0)[None, :]
            rows.append(norm_rows(m.T))
        return rows

    protoN = norm_rows(proto_ref[...])
    AS = jnp.concatenate(mean_rows(sS_ref, cS) + [protoN], axis=0)
    AT = jnp.concatenate(mean_rows(sT_ref, cT) + [protoN], axis=0)

    nt = (((1,), (1,)), ((), ()))
    ES = jnp.exp(jax.lax.dot_general(AS, AS, nt, precision=_HI)
                 / TEMPERATURE)
    ET = jnp.exp(jax.lax.dot_general(AT, AS, nt, precision=_HI)
                 / TEMPERATURE)

    presS = (cS > 0).astype(jnp.float32)
    presT = (cT > 0).astype(jnp.float32)
    ones21 = jnp.ones((1, nc1), jnp.float32)
    validS = jnp.concatenate([presS, ones21], axis=0)
    validT = jnp.concatenate([presT, ones21], axis=0)
    cS_tot = 1.0 + jnp.sum(presS, axis=0, keepdims=True)
    cT_tot = 1.0 + jnp.sum(presT, axis=0, keepdims=True)

    Eb = (jax.lax.broadcasted_iota(jnp.int32, (NR, NS), 0) // nc1
          == jax.lax.broadcasted_iota(jnp.int32, (NR, NS), 1)
          ).astype(jnp.float32)
    Ec = (jax.lax.broadcasted_iota(jnp.int32, (NR, nc1), 0) % nc1
          == jax.lax.broadcasted_iota(jnp.int32, (NR, nc1), 1)
          ).astype(jnp.float32)
    Fb = (jax.lax.broadcasted_iota(jnp.int32, (NS, NR), 1) // nc1
          == jax.lax.broadcasted_iota(jnp.int32, (NS, NR), 0)
          ).astype(jnp.float32)
    Fc = (jax.lax.broadcasted_iota(jnp.int32, (nc1, NR), 1) % nc1
          == jax.lax.broadcasted_iota(jnp.int32, (nc1, NR), 0)
          ).astype(jnp.float32)

    def row_lut(tab):
        return jnp.sum(jax.lax.dot(Eb, tab, precision=_HI) * Ec,
                       axis=1, keepdims=True)

    def col_lut(tab):
        return jnp.sum(jax.lax.dot(tab, Fc, precision=_HI) * Fb,
                       axis=0, keepdims=True)

    ri = jax.lax.broadcasted_iota(jnp.int32, (NR, NR), 0)
    ci = jax.lax.broadcasted_iota(jnp.int32, (NR, NR), 1)
    rc = ri % nc1
    cc = ci % nc1

    w_row = row_lut(validS / cS_tot)
    mrow = ((rc >= 1) & (rc != cc)
            & (rc.astype(jnp.float32) <= nct)).astype(jnp.float32)
    snt = jnp.sum(ES * w_row * mrow, axis=0, keepdims=True)

    vT_row = row_lut(validT)
    bc = jnp.broadcast_to
    cTtot_col = col_lut(bc(cT_tot, (NS, nc1)))
    dj_col = col_lut(bc(jnp.maximum(cS_tot - 1.0, 1.0), (NS, nc1)))
    fval_col = col_lut(jnp.concatenate(
        [presS, jnp.zeros((1, nc1), jnp.float32)], axis=0))
    ci1 = jax.lax.broadcasted_iota(jnp.int32, (1, NR), 1)
    ccol = ci1 % nc1
    fmask = fval_col * ((ccol >= 1) & (ccol <= noc)
                        & (ccol.astype(jnp.float32) <= noct)
                        ).astype(jnp.float32)

    terms = jnp.log(ET / (snt + ET))
    pmask = vT_row * (rc == cc).astype(jnp.float32)
    loss = -jnp.sum(terms * pmask * (fmask / cTtot_col / dj_col),
                    keepdims=True)

    ci21 = jax.lax.broadcasted_iota(jnp.int32, (1, nc1), 1)
    emask = ((ci21 >= 1) & (ci21 <= noc)
             & (ci21.astype(jnp.float32) <= noct)
             & (cS_tot > 1.0)).astype(jnp.float32)
    cnt_exist = jnp.sum(emask, keepdims=True)
    loss_ref[...] = jnp.where(cnt_exist > 0.0, loss / cnt_exist, loss)


def kernel(labels, features_old, features, outputs_old, prototypes,
           num_class, num_old_class):
    B, C, H, W = features.shape
    nch = outputs_old.shape[1]
    noc = nch - 1
    nc1 = prototypes.shape[0]
    NPIX = H * W
    R = 64
    CHUNK = 128 * R
    K = NPIX // CHUNK

    lab_v = labels.astype(jnp.int32)
    fT = features_old.reshape(B, C, NPIX)
    fS = features.reshape(B, C, NPIX)
    scal = jnp.stack([jnp.asarray(num_class, jnp.float32),
                      jnp.asarray(num_old_class, jnp.float32)]).reshape(1, 2)

    sT, cT, sS, cS = pl.pallas_call(
        functools.partial(_stage1_kernel, nch, nc1, R),
        grid=(B, K),
        in_specs=[
            pl.BlockSpec((1, 4 * R, 512), lambda b, p: (b, p, 0)),
            pl.BlockSpec((1, nch, 4 * R, 512), lambda b, p: (b, 0, p, 0)),
            pl.BlockSpec((1, C, CHUNK), lambda b, p: (b, 0, p)),
            pl.BlockSpec((1, C, CHUNK), lambda b, p: (b, 0, p)),
        ],
        out_specs=[
            pl.BlockSpec((1, C, nc1), lambda b, p: (b, 0, 0)),
            pl.BlockSpec((1, 1, nc1), lambda b, p: (b, 0, 0)),
            pl.BlockSpec((1, C, nc1), lambda b, p: (b, 0, 0)),
            pl.BlockSpec((1, 1, nc1), lambda b, p: (b, 0, 0)),
        ],
        out_shape=[
            jax.ShapeDtypeStruct((B, C, nc1), jnp.float32),
            jax.ShapeDtypeStruct((B, 1, nc1), jnp.float32),
            jax.ShapeDtypeStruct((B, C, nc1), jnp.float32),
            jax.ShapeDtypeStruct((B, 1, nc1), jnp.float32),
        ],
    )(lab_v, outputs_old, fT, fS)

    loss = pl.pallas_call(
        functools.partial(_stage2_kernel, nc1, noc),
        in_specs=[
            pl.BlockSpec((B, C, nc1), lambda: (0, 0, 0)),
            pl.BlockSpec((B, 1, nc1), lambda: (0, 0, 0)),
            pl.BlockSpec((B, C, nc1), lambda: (0, 0, 0)),
            pl.BlockSpec((B, 1, nc1), lambda: (0, 0, 0)),
            pl.BlockSpec((nc1, C), lambda: (0, 0)),
            pl.BlockSpec((1, 2), lambda: (0, 0)),
        ],
        out_specs=pl.BlockSpec((1, 1), lambda: (0, 0)),
        out_shape=jax.ShapeDtypeStruct((1, 1), jnp.float32),
    )(sT, cT, sS, cS, prototypes, scal)

    return loss[0, 0]
